# Initial kernel scaffold; baseline (speedup 1.0000x reference)
import jax
import jax.numpy as jnp
from jax.experimental import pallas as pl


def kernel(inputs, table):
    raise NotImplementedError("write your pallas kernel here")


# speedup vs baseline: 163.4180x; 163.4180x over previous
import functools

import jax
import jax.numpy as jnp
from jax import lax
from jax.experimental import pallas as pl
from jax.experimental.pallas import tpu as pltpu
from jax.experimental.pallas import tpu_sc as plsc

_TABLE = 120000
_B = 16384
_H = 200
_TOT = _B * _H
_NW = 32
_PER_W = _TOT // _NW
_CHUNK = 4096
_NCHUNK = _PER_W // _CHUNK

_mesh = plsc.VectorSubcoreMesh(core_axis_name="c", subcore_axis_name="s")


@functools.partial(
    pl.kernel,
    mesh=_mesh,
    out_type=jax.ShapeDtypeStruct((_TOT,), jnp.int32),
    compiler_params=pltpu.CompilerParams(needs_layout_passes=False),
    scratch_types=[
        pltpu.VMEM((_TABLE,), jnp.int32),
        pltpu.VMEM((_CHUNK,), jnp.int32),
        pltpu.VMEM((_CHUNK,), jnp.int32),
    ],
)
def _sc_gather(idx_hbm, table_hbm, out_hbm, table_v, idx_v, out_v):
    wid = lax.axis_index("s") * 2 + lax.axis_index("c")
    base = wid * _PER_W
    pltpu.sync_copy(table_hbm, table_v)

    def chunk_body(c, carry):
        off = base + c * _CHUNK
        pltpu.sync_copy(idx_hbm.at[pl.ds(off, _CHUNK)], idx_v)

        def vec_body(i, carry2):
            v = idx_v[pl.ds(i * 16, 16)]
            out_v[pl.ds(i * 16, 16)] = plsc.load_gather(table_v, [v])
            return carry2

        lax.fori_loop(0, _CHUNK // 16, vec_body, 0, unroll=8)
        pltpu.sync_copy(out_v, out_hbm.at[pl.ds(off, _CHUNK)])
        return carry

    lax.fori_loop(0, _NCHUNK, chunk_body, 0)


def kernel(inputs, table):
    idx = inputs.astype(jnp.int32).reshape(_TOT)
    out = _sc_gather(idx, table)
    return out.reshape(_B, _H)


# speedup vs baseline: 224.0929x; 1.3713x over previous
import functools

import jax
import jax.numpy as jnp
from jax import lax
from jax.experimental import pallas as pl
from jax.experimental.pallas import tpu as pltpu
from jax.experimental.pallas import tpu_sc as plsc

_TABLE = 120000
_B = 16384
_H = 200
_TOT = _B * _H
_NW = 32
_PER_W = _TOT // _NW
_NB = 4
_C = 2560
_NCHUNK = _PER_W // _C
_NGRP = _NCHUNK // _NB

_mesh = plsc.VectorSubcoreMesh(core_axis_name="c", subcore_axis_name="s")


@functools.partial(
    pl.kernel,
    mesh=_mesh,
    out_type=jax.ShapeDtypeStruct((_TOT,), jnp.int32),
    compiler_params=pltpu.CompilerParams(needs_layout_passes=False),
    scratch_types=[
        pltpu.VMEM((_TABLE,), jnp.int32),
        [pltpu.VMEM((_C,), jnp.int32)] * _NB,
        [pltpu.SemaphoreType.DMA] * _NB,
        [pltpu.SemaphoreType.DMA] * _NB,
    ],
)
def _sc_gather(idx_hbm, table_hbm, out_hbm, table_v, bufs, sem_in, sem_out):
    wid = lax.axis_index("s") * 2 + lax.axis_index("c")
    base = wid * _PER_W

    def in_copy(c, j):
        return pltpu.make_async_copy(
            idx_hbm.at[pl.ds(base + c * _C, _C)], bufs[j], sem_in[j])

    def out_copy(c, j):
        return pltpu.make_async_copy(
            bufs[j], out_hbm.at[pl.ds(base + c * _C, _C)], sem_out[j])

    in_copy(0, 0).start()
    pltpu.sync_copy(table_hbm, table_v)

    def group(g, carry):
        for j in range(_NB):
            c = g * _NB + j
            jn = (j + 1) % _NB

            @pl.when(c >= _NB - 1)
            def _():
                out_copy(c - (_NB - 1), jn).wait()

            @pl.when(c + 1 < _NCHUNK)
            def _():
                in_copy(c + 1, jn).start()

            in_copy(c, j).wait()

            def vec(i, carry2):
                v = bufs[j][pl.ds(i * 16, 16)]
                bufs[j][pl.ds(i * 16, 16)] = plsc.load_gather(table_v, [v])
                return carry2

            lax.fori_loop(0, _C // 16, vec, 0, unroll=8)
            out_copy(c, j).start()
        return carry

    lax.fori_loop(0, _NGRP, group, 0)

    for c in range(_NCHUNK - (_NB - 1), _NCHUNK):
        out_copy(c, c % _NB).wait()


def kernel(inputs, table):
    idx = inputs.astype(jnp.int32).reshape(_TOT)
    out = _sc_gather(idx, table)
    return out.reshape(_B, _H)


# speedup vs baseline: 320.7757x; 1.4314x over previous
import functools

import jax
import jax.numpy as jnp
from jax import lax
from jax.experimental import pallas as pl
from jax.experimental.pallas import tpu as pltpu
from jax.experimental.pallas import tpu_sc as plsc

_TABLE = 120000
_B = 16384
_H = 200
_NW = 32
_ROWS_W = _B // _NW
_R = 8
_NB = 4
_NCHUNK = _ROWS_W // _R
_NGRP = _NCHUNK // _NB
_NVEC = _H // 16

_mesh = plsc.VectorSubcoreMesh(core_axis_name="c", subcore_axis_name="s")


@functools.partial(
    pl.kernel,
    mesh=_mesh,
    out_type=jax.ShapeDtypeStruct((_B, _H), jnp.int32),
    compiler_params=pltpu.CompilerParams(needs_layout_passes=False),
    scratch_types=[
        pltpu.VMEM((_TABLE,), jnp.int32),
        [pltpu.VMEM((_R, _H), jnp.int32)] * _NB,
        [pltpu.SemaphoreType.DMA] * _NB,
        [pltpu.SemaphoreType.DMA] * _NB,
    ],
)
def _sc_gather(idx_hbm, table_hbm, out_hbm, table_v, bufs, sem_in, sem_out):
    wid = lax.axis_index("s") * 2 + lax.axis_index("c")
    row0 = wid * _ROWS_W

    def in_copy(c, j):
        return pltpu.make_async_copy(
            idx_hbm.at[pl.ds(row0 + c * _R, _R), :], bufs[j], sem_in[j])

    def out_copy(c, j):
        return pltpu.make_async_copy(
            bufs[j], out_hbm.at[pl.ds(row0 + c * _R, _R), :], sem_out[j])

    in_copy(0, 0).start()
    pltpu.sync_copy(table_hbm, table_v)

    def group(g, carry):
        for j in range(_NB):
            c = g * _NB + j
            jn = (j + 1) % _NB

            @pl.when(c >= _NB - 1)
            def _():
                out_copy(c - (_NB - 1), jn).wait()

            @pl.when(c + 1 < _NCHUNK)
            def _():
                in_copy(c + 1, jn).start()

            in_copy(c, j).wait()

            def row_body(r, carry2):
                vt = bufs[j][r, pl.ds(_H - 16, 16)]
                for k in range(_NVEC):
                    v = bufs[j][r, pl.ds(16 * k, 16)]
                    bufs[j][r, pl.ds(16 * k, 16)] = plsc.load_gather(
                        table_v, [v])
                bufs[j][r, pl.ds(_H - 16, 16)] = plsc.load_gather(
                    table_v, [vt])
                return carry2

            lax.fori_loop(0, _R, row_body, 0, unroll=2)
            out_copy(c, j).start()
        return carry

    lax.fori_loop(0, _NGRP, group, 0)

    for c in range(_NCHUNK - (_NB - 1), _NCHUNK):
        out_copy(c, c % _NB).wait()


def kernel(inputs, table):
    return _sc_gather(inputs.astype(jnp.int32), table)


# speedup vs baseline: 347.1630x; 1.0823x over previous
import functools

import jax
import jax.numpy as jnp
from jax import lax
from jax.experimental import pallas as pl
from jax.experimental.pallas import tpu as pltpu
from jax.experimental.pallas import tpu_sc as plsc

_TABLE = 120000
_B = 16384
_H = 200
_NW = 32
_ROWS_W = _B // _NW
_R = 8
_NB = 4
_NCHUNK = _ROWS_W // _R
_NGRP = _NCHUNK // _NB
_NVEC = _H // 16

_mesh = plsc.VectorSubcoreMesh(core_axis_name="c", subcore_axis_name="s")


@functools.partial(
    pl.kernel,
    mesh=_mesh,
    out_type=jax.ShapeDtypeStruct((_B, _H), jnp.int32),
    compiler_params=pltpu.CompilerParams(needs_layout_passes=False),
    scratch_types=[
        pltpu.VMEM((_TABLE,), jnp.int32),
        [pltpu.VMEM((_R, _H), jnp.int32)] * _NB,
        [pltpu.SemaphoreType.DMA] * _NB,
        [pltpu.SemaphoreType.DMA] * _NB,
    ],
)
def _sc_gather(idx_hbm, table_hbm, out_hbm, table_v, bufs, sem_in, sem_out):
    wid = lax.axis_index("s") * 2 + lax.axis_index("c")
    row0 = wid * _ROWS_W

    def in_copy(c, j):
        return pltpu.make_async_copy(
            idx_hbm.at[pl.ds(row0 + c * _R, _R), :], bufs[j], sem_in[j])

    def out_copy(c, j):
        return pltpu.make_async_copy(
            bufs[j], out_hbm.at[pl.ds(row0 + c * _R, _R), :], sem_out[j])

    in_copy(0, 0).start()
    pltpu.sync_copy(table_hbm, table_v)

    def group(g, carry):
        for j in range(_NB):
            c = g * _NB + j
            jn = (j + 1) % _NB

            @pl.when(c >= _NB - 1)
            def _():
                out_copy(c - (_NB - 1), jn).wait()

            @pl.when(c + 1 < _NCHUNK)
            def _():
                in_copy(c + 1, jn).start()

            in_copy(c, j).wait()

            @plsc.parallel_loop(0, _R, unroll=4)
            def row_body(r):
                vt = bufs[j][r, pl.ds(_H - 16, 16)]
                for k in range(_NVEC):
                    v = bufs[j][r, pl.ds(16 * k, 16)]
                    bufs[j][r, pl.ds(16 * k, 16)] = plsc.load_gather(
                        table_v, [v])
                bufs[j][r, pl.ds(_H - 16, 16)] = plsc.load_gather(
                    table_v, [vt])
            out_copy(c, j).start()
        return carry

    lax.fori_loop(0, _NGRP, group, 0)

    for c in range(_NCHUNK - (_NB - 1), _NCHUNK):
        out_copy(c, c % _NB).wait()


def kernel(inputs, table):
    return _sc_gather(inputs.astype(jnp.int32), table)


# speedup vs baseline: 550.5606x; 1.5859x over previous
import functools

import jax
import jax.numpy as jnp
from jax import lax
from jax.experimental import pallas as pl
from jax.experimental.pallas import tpu as pltpu
from jax.experimental.pallas import tpu_sc as plsc

_TABLE = 120000
_B = 16384
_H = 200
_NW = 32
_C = 2048
_CPR = _B // _C
_NCHUNK = _H * _CPR // _NW
_NB = 5
_NGRP = _NCHUNK // _NB

_mesh = plsc.VectorSubcoreMesh(core_axis_name="c", subcore_axis_name="s")


@functools.partial(
    pl.kernel,
    mesh=_mesh,
    out_type=jax.ShapeDtypeStruct((_H, _B), jnp.int32),
    compiler_params=pltpu.CompilerParams(needs_layout_passes=False),
    scratch_types=[
        pltpu.VMEM((_TABLE,), jnp.int32),
        [pltpu.VMEM((1, _C), jnp.int32)] * _NB,
        [pltpu.SemaphoreType.DMA] * _NB,
        [pltpu.SemaphoreType.DMA] * _NB,
    ],
)
def _sc_gather(idx_hbm, table_hbm, out_hbm, table_v, bufs, sem_in, sem_out):
    wid = lax.axis_index("s") * 2 + lax.axis_index("c")
    k0 = wid * _NCHUNK

    def in_copy(c, j):
        k = k0 + c
        return pltpu.make_async_copy(
            idx_hbm.at[pl.ds(k // _CPR, 1), pl.ds((k % _CPR) * _C, _C)],
            bufs[j], sem_in[j])

    def out_copy(c, j):
        k = k0 + c
        return pltpu.make_async_copy(
            bufs[j],
            out_hbm.at[pl.ds(k // _CPR, 1), pl.ds((k % _CPR) * _C, _C)],
            sem_out[j])

    in_copy(0, 0).start()
    pltpu.sync_copy(table_hbm, table_v)

    def group(g, carry):
        for j in range(_NB):
            c = g * _NB + j
            jn = (j + 1) % _NB

            @pl.when(c >= _NB - 1)
            def _():
                out_copy(c - (_NB - 1), jn).wait()

            @pl.when(c + 1 < _NCHUNK)
            def _():
                in_copy(c + 1, jn).start()

            in_copy(c, j).wait()

            @plsc.parallel_loop(0, _C // 16, unroll=4)
            def vec_body(i):
                v = bufs[j][0, pl.ds(i * 16, 16)]
                bufs[j][0, pl.ds(i * 16, 16)] = plsc.load_gather(
                    table_v, [v])

            out_copy(c, j).start()
        return carry

    lax.fori_loop(0, _NGRP, group, 0)

    for c in range(_NCHUNK - (_NB - 1), _NCHUNK):
        out_copy(c, c % _NB).wait()


def kernel(inputs, table):
    idx_t = jnp.transpose(inputs.astype(jnp.int32))
    return jnp.transpose(_sc_gather(idx_t, table))


# speedup vs baseline: 679.3880x; 1.2340x over previous
import functools

import jax
import jax.numpy as jnp
from jax import lax
from jax.experimental import pallas as pl
from jax.experimental.pallas import tpu as pltpu
from jax.experimental.pallas import tpu_sc as plsc

_TABLE = 120000
_B = 16384
_H = 200
_NW = 32
_C = 2048
_CPR = _B // _C
_NCHUNK = _H * _CPR // _NW
_NB = 5
_NGRP = _NCHUNK // _NB

_mesh = plsc.VectorSubcoreMesh(core_axis_name="c", subcore_axis_name="s")


@functools.partial(
    pl.kernel,
    mesh=_mesh,
    out_type=jax.ShapeDtypeStruct((_H, _B), jnp.int32),
    compiler_params=pltpu.CompilerParams(needs_layout_passes=False),
    scratch_types=[
        pltpu.VMEM((_TABLE,), jnp.int32),
        [pltpu.VMEM((1, _C), jnp.int32)] * _NB,
        [pltpu.SemaphoreType.DMA] * _NB,
        [pltpu.SemaphoreType.DMA] * _NB,
    ],
)
def _sc_gather(idx_hbm, table_hbm, out_hbm, table_v, bufs, sem_in, sem_out):
    wid = lax.axis_index("s") * 2 + lax.axis_index("c")
    k0 = wid * _NCHUNK

    def in_copy(c, j):
        k = k0 + c
        return pltpu.make_async_copy(
            idx_hbm.at[pl.ds(k // _CPR, 1), pl.ds((k % _CPR) * _C, _C)],
            bufs[j], sem_in[j])

    def out_copy(c, j):
        k = k0 + c
        return pltpu.make_async_copy(
            bufs[j],
            out_hbm.at[pl.ds(k // _CPR, 1), pl.ds((k % _CPR) * _C, _C)],
            sem_out[j])

    _DEPTH = _NB - 2

    for c in range(_DEPTH):
        in_copy(c, c).start()
    pltpu.sync_copy(table_hbm, table_v)

    def group(g, carry):
        for j in range(_NB):
            c = g * _NB + j
            jp = (j + _DEPTH) % _NB

            @pl.when(c >= 2)
            def _():
                out_copy(c - 2, jp).wait()

            @pl.when(c + _DEPTH < _NCHUNK)
            def _():
                in_copy(c + _DEPTH, jp).start()

            in_copy(c, j).wait()

            @plsc.parallel_loop(0, _C // 16, unroll=4)
            def vec_body(i):
                v = bufs[j][0, pl.ds(i * 16, 16)]
                bufs[j][0, pl.ds(i * 16, 16)] = plsc.load_gather(
                    table_v, [v])

            out_copy(c, j).start()
        return carry

    lax.fori_loop(0, _NGRP, group, 0)

    for c in range(_NCHUNK - 2, _NCHUNK):
        out_copy(c, c % _NB).wait()


def kernel(inputs, table):
    idx_t = jnp.transpose(inputs.astype(jnp.int32))
    return jnp.transpose(_sc_gather(idx_t, table))


# speedup vs baseline: 680.8338x; 1.0021x over previous
import functools

import jax
import jax.numpy as jnp
from jax import lax
from jax.experimental import pallas as pl
from jax.experimental.pallas import tpu as pltpu
from jax.experimental.pallas import tpu_sc as plsc

_TABLE = 120000
_B = 16384
_H = 200
_NW = 32
_C = 2048
_CPR = _B // _C
_NCHUNK = _H * _CPR // _NW
_NB = 5
_NGRP = _NCHUNK // _NB

_mesh = plsc.VectorSubcoreMesh(core_axis_name="c", subcore_axis_name="s")


@functools.partial(
    pl.kernel,
    mesh=_mesh,
    out_type=jax.ShapeDtypeStruct((_H, _B), jnp.int32),
    compiler_params=pltpu.CompilerParams(needs_layout_passes=False),
    scratch_types=[
        pltpu.VMEM((_TABLE,), jnp.int32),
        [pltpu.VMEM((1, _C), jnp.int32)] * _NB,
        [pltpu.SemaphoreType.DMA] * _NB,
        [pltpu.SemaphoreType.DMA] * _NB,
    ],
)
def _sc_gather(idx_hbm, table_hbm, out_hbm, table_v, bufs, sem_in, sem_out):
    wid = lax.axis_index("s") * 2 + lax.axis_index("c")
    k0 = wid * _NCHUNK

    def in_copy(c, j):
        k = k0 + c
        return pltpu.make_async_copy(
            idx_hbm.at[pl.ds(k // _CPR, 1), pl.ds((k % _CPR) * _C, _C)],
            bufs[j], sem_in[j])

    def out_copy(c, j):
        k = k0 + c
        return pltpu.make_async_copy(
            bufs[j],
            out_hbm.at[pl.ds(k // _CPR, 1), pl.ds((k % _CPR) * _C, _C)],
            sem_out[j])

    _DEPTH = _NB - 2

    for c in range(_DEPTH):
        in_copy(c, c).start()
    pltpu.sync_copy(table_hbm, table_v)

    def group(g, carry):
        for j in range(_NB):
            c = g * _NB + j
            jp = (j + _DEPTH) % _NB

            @pl.when(c >= 2)
            def _():
                out_copy(c - 2, jp).wait()

            @pl.when(c + _DEPTH < _NCHUNK)
            def _():
                in_copy(c + _DEPTH, jp).start()

            in_copy(c, j).wait()

            @plsc.parallel_loop(0, _C // 16, unroll=8)
            def vec_body(i):
                v = bufs[j][0, pl.ds(i * 16, 16)]
                bufs[j][0, pl.ds(i * 16, 16)] = plsc.load_gather(
                    table_v, [v])

            out_copy(c, j).start()
        return carry

    lax.fori_loop(0, _NGRP, group, 0)

    for c in range(_NCHUNK - 2, _NCHUNK):
        out_copy(c, c % _NB).wait()


def kernel(inputs, table):
    idx_t = jnp.transpose(inputs.astype(jnp.int32))
    return jnp.transpose(_sc_gather(idx_t, table))
